# baseline (speedup 1.0000x reference)
import functools

import jax
import jax.numpy as jnp
from jax import lax
from jax.experimental import pallas as pl
from jax.experimental.pallas import tpu as pltpu
from jax.experimental.pallas import tpu_sc as plsc

N = 10000
N_PAD = 10240
E = 160000
H = 256
HALF = 128
NUM_LAYERS = 6
STEP_LAYER = 3
EPS = 1e-5

NSUB = 16
EDGES_PER_TILE = E // NSUB
EBATCH = 104
EPT_PAD = 10192
NEB = EPT_PAD // EBATCH
ROWS_PER_TILE = N_PAD // NSUB

ROWB = 1000
NRB = N // ROWB

@functools.cache
def _mesh():
    return plsc.VectorSubcoreMesh(
        core_axis_name="c", subcore_axis_name="s",
        num_cores=2, num_subcores=NSUB)


def _deg_body(dst_hbm, deg_hbm, dstv, degv, tmpv, outv, shpart):
    c = lax.axis_index("c")
    s = lax.axis_index("s")

    @pl.when(c == 0)
    def _():
        zeros = jnp.zeros((16,), jnp.float32)
        ones = jnp.ones((16,), jnp.float32)

        def zstep(i, carry):
            degv[pl.ds(i * 16, 16)] = zeros
            return carry

        lax.fori_loop(0, N_PAD // 16, zstep, 0)
        pltpu.sync_copy(dst_hbm.at[s], dstv)

        def cstep(k, carry):
            idx = dstv[pl.ds(k * 16, 16)]
            plsc.addupdate_scatter(degv, [idx], ones)
            return carry

        lax.fori_loop(0, EDGES_PER_TILE // 16, cstep, 0)
        pltpu.sync_copy(degv, shpart.at[s])
        plsc.subcore_barrier()

        cslc = pl.ds(s * ROWS_PER_TILE, ROWS_PER_TILE)
        pltpu.sync_copy(shpart.at[0, cslc], outv)
        for r in range(1, NSUB):
            pltpu.sync_copy(shpart.at[r, cslc], tmpv)

            def astep(q, carry):
                sl = pl.ds(q * 16, 16)
                outv[sl] = outv[sl] + tmpv[sl]
                return carry

            lax.fori_loop(0, ROWS_PER_TILE // 16, astep, 0)
        pltpu.sync_copy(outv, deg_hbm.at[cslc])


@functools.cache
def _deg_kernel():
    return pl.kernel(
        _deg_body,
        out_type=jax.ShapeDtypeStruct((N_PAD,), jnp.float32),
        mesh=_mesh(),
        scratch_types=[
            pltpu.VMEM((EDGES_PER_TILE,), jnp.int32),
            pltpu.VMEM((N_PAD,), jnp.float32),
            pltpu.VMEM((ROWS_PER_TILE,), jnp.float32),
            pltpu.VMEM((ROWS_PER_TILE,), jnp.float32),
            pltpu.VMEM_SHARED((NSUB, N_PAD), jnp.float32),
        ],
        compiler_params=pltpu.CompilerParams(needs_layout_passes=False),
    )


def _scatter_half(h_hbm, dst_hbm, acc_hbm, srci_v, dstc_v, rows_a, rows_b,
                  sg, si, ss, shacc, s):
    rslc = pl.ds(s * ROWS_PER_TILE, ROWS_PER_TILE)
    pltpu.sync_copy(h_hbm.at[rslc], shacc.at[rslc])
    plsc.subcore_barrier()

    def gath(j, buf):
        return pltpu.async_copy(
            h_hbm.at[srci_v.at[pl.ds(j * EBATCH, EBATCH)]], buf, sg)

    def gath_wait(j, buf):
        pltpu.make_async_copy(
            h_hbm.at[srci_v.at[pl.ds(j * EBATCH, EBATCH)]], buf, sg).wait()

    def idx(j, p):
        return pltpu.async_copy(dst_hbm.at[s, j], dstc_v.at[p], si)

    def idx_wait(j, p):
        pltpu.make_async_copy(dst_hbm.at[s, j], dstc_v.at[p], si).wait()

    def sct(buf, p):
        return pltpu.async_copy(buf, shacc.at[dstc_v.at[p]], ss, add=True)

    def sct_wait(buf, p):
        pltpu.make_async_copy(buf, shacc.at[dstc_v.at[p]], ss).wait()

    idx(0, 0)
    gath(0, rows_a)
    idx(1, 1)
    gath(1, rows_b)

    def step2(k, carry):
        j = 2 * k
        gath_wait(j, rows_a)
        idx_wait(j, 0)
        sct(rows_a, 0)
        sct_wait(rows_a, 0)

        @pl.when(j + 2 < NEB)
        def _():
            gath(j + 2, rows_a)
            idx(j + 2, 0)

        gath_wait(j + 1, rows_b)
        idx_wait(j + 1, 1)
        sct(rows_b, 1)
        sct_wait(rows_b, 1)

        @pl.when(j + 3 < NEB)
        def _():
            gath(j + 3, rows_b)
            idx(j + 3, 1)

        return carry

    lax.fori_loop(0, NEB // 2, step2, 0)
    plsc.subcore_barrier()
    pltpu.sync_copy(shacc.at[rslc], acc_hbm.at[rslc])


def _gcn_scatter_body(h0, h1, src_hbm, dst_hbm, acc0, acc1,
                      srci_v, dstc_v, rows_a, rows_b, sg, si, ss, shacc):
    c = lax.axis_index("c")
    s = lax.axis_index("s")
    pltpu.sync_copy(src_hbm.at[s], srci_v)

    @pl.when(c == 0)
    def _():
        _scatter_half(h0, dst_hbm, acc0, srci_v, dstc_v, rows_a, rows_b,
                      sg, si, ss, shacc, s)

    @pl.when(c == 1)
    def _():
        _scatter_half(h1, dst_hbm, acc1, srci_v, dstc_v, rows_a, rows_b,
                      sg, si, ss, shacc, s)


@functools.cache
def _gcn_scatter():
    return pl.kernel(
        _gcn_scatter_body,
        out_type=(
            jax.ShapeDtypeStruct((N_PAD, HALF), jnp.float32),
            jax.ShapeDtypeStruct((N_PAD, HALF), jnp.float32),
        ),
        mesh=_mesh(),
        scratch_types=[
            pltpu.VMEM((EPT_PAD,), jnp.int32),
            pltpu.VMEM((2, EBATCH), jnp.int32),
            pltpu.VMEM((EBATCH, HALF), jnp.float32),
            pltpu.VMEM((EBATCH, HALF), jnp.float32),
            pltpu.SemaphoreType.DMA,
            pltpu.SemaphoreType.DMA,
            pltpu.SemaphoreType.DMA,
            pltpu.VMEM_SHARED((N_PAD, HALF), jnp.float32),
        ],
    )


def _mm_body(x_ref, w_ref, deg_ref, h0_ref, h1_ref):
    deg = deg_ref[...] + 1.0
    dinv = 1.0 / jnp.sqrt(deg)
    h = jnp.dot(x_ref[...], w_ref[...], preferred_element_type=jnp.float32)
    h = h * dinv
    h0_ref[...] = h[:, :HALF]
    h1_ref[...] = h[:, HALF:]


def _mm_call(x, w, deg16):
    return pl.pallas_call(
        _mm_body,
        grid=(NRB,),
        in_specs=[
            pl.BlockSpec((ROWB, H), lambda j: (j, 0)),
            pl.BlockSpec((H, H), lambda j: (0, 0)),
            pl.BlockSpec((ROWB, 1), lambda j: (j, 0)),
        ],
        out_specs=(
            pl.BlockSpec((ROWB, HALF), lambda j: (j, 0)),
            pl.BlockSpec((ROWB, HALF), lambda j: (j, 0)),
        ),
        out_shape=(
            jax.ShapeDtypeStruct((N_PAD, HALF), jnp.float32),
            jax.ShapeDtypeStruct((N_PAD, HALF), jnp.float32),
        ),
    )(x, w, deg16)


def _post_body(has_res, a0_ref, a1_ref, deg_ref, b_ref, g_ref, be_ref,
               *rest):
    if has_res:
        res_ref, y_ref, sum_ref, sq_ref = rest
    else:
        y_ref, sum_ref, sq_ref = rest
        res_ref = None
    j = pl.program_id(0)

    @pl.when(j == 0)
    def _():
        sum_ref[...] = jnp.zeros_like(sum_ref)
        sq_ref[...] = jnp.zeros_like(sq_ref)

    def compute_z():
        a = jnp.concatenate([a0_ref[...], a1_ref[...]], axis=1)
        dinv = 1.0 / jnp.sqrt(deg_ref[...] + 1.0)
        z = jnp.maximum(a * dinv + b_ref[...], 0.0)
        if has_res:
            z = jnp.maximum(z + res_ref[...], 0.0)
        return z

    @pl.when(j < NRB)
    def _():
        z = compute_z()
        sum_ref[...] += jnp.sum(z, axis=0, keepdims=True)
        sq_ref[...] += jnp.sum(z * z, axis=0, keepdims=True)

    @pl.when(j >= NRB)
    def _():
        z = compute_z()
        mean = sum_ref[...] * (1.0 / N)
        var = sq_ref[...] * (1.0 / N) - mean * mean
        scale = g_ref[...] / jnp.sqrt(var + EPS)
        y_ref[...] = (z - mean) * scale + be_ref[...]


def _post_call(a0, a1, deg2d, bvec, gvec, bevec, res):
    has_res = res is not None
    in_specs = [
        pl.BlockSpec((ROWB, HALF), lambda j: (j % NRB, 0)),
        pl.BlockSpec((ROWB, HALF), lambda j: (j % NRB, 0)),
        pl.BlockSpec((ROWB, 1), lambda j: (j % NRB, 0)),
        pl.BlockSpec((1, H), lambda j: (0, 0)),
        pl.BlockSpec((1, H), lambda j: (0, 0)),
        pl.BlockSpec((1, H), lambda j: (0, 0)),
    ]
    args = [a0, a1, deg2d, bvec, gvec, bevec]
    if has_res:
        in_specs.append(pl.BlockSpec((ROWB, H), lambda j: (j % NRB, 0)))
        args.append(res)
    return pl.pallas_call(
        functools.partial(_post_body, has_res),
        grid=(2 * NRB,),
        in_specs=in_specs,
        out_specs=pl.BlockSpec((ROWB, H), lambda j: (j % NRB, 0)),
        out_shape=jax.ShapeDtypeStruct((N, H), jnp.float32),
        scratch_shapes=[
            pltpu.VMEM((1, H), jnp.float32),
            pltpu.VMEM((1, H), jnp.float32),
        ],
    )(*args)


def kernel(x, edge_index, W, b, gamma, beta):
    ei = edge_index.astype(jnp.int32)
    pad = EPT_PAD - EDGES_PER_TILE
    src = jnp.pad(ei[0].reshape(NSUB, EDGES_PER_TILE), ((0, 0), (0, pad)),
                  constant_values=0)
    pad_dst = N + (jnp.arange(pad, dtype=jnp.int32)[None, :]
                   + 17 * jnp.arange(NSUB, dtype=jnp.int32)[:, None]) % (N_PAD - N)
    dst = jnp.concatenate(
        [ei[1].reshape(NSUB, EDGES_PER_TILE), pad_dst], axis=1
    ).reshape(NSUB, NEB, EBATCH)

    dst_flat = ei[1].reshape(NSUB, EDGES_PER_TILE)
    deg2d = _deg_kernel()(dst_flat).reshape(N_PAD, 1)

    x0 = x
    cur = x
    for i in range(NUM_LAYERS):
        if i % STEP_LAYER == 0:
            x0 = cur
        x_in = cur
        h0, h1 = _mm_call(cur, W[i], deg2d)
        a0, a1 = _gcn_scatter()(h0, h1, src, dst)
        if i == 0:
            res = None
        elif (i != NUM_LAYERS - 1) and ((i + 1) % STEP_LAYER != 0):
            res = x_in
        else:
            res = x0
        cur = _post_call(a0, a1, deg2d, b[i].reshape(1, H),
                         gamma[i].reshape(1, H), beta[i].reshape(1, H), res)
    return cur


# speedup vs baseline: 1.5172x; 1.5172x over previous
import functools

import jax
import jax.numpy as jnp
from jax import lax
from jax.experimental import pallas as pl
from jax.experimental.pallas import tpu as pltpu
from jax.experimental.pallas import tpu_sc as plsc

N = 10000
N_PAD = 10240
E = 160000
H = 256
HALF = 128
NUM_LAYERS = 6
STEP_LAYER = 3
EPS = 1e-5

NSUB = 16
EDGES_PER_TILE = E // NSUB
EBATCH = 88
EPT_PAD = 10032
NEB = EPT_PAD // EBATCH
ROWS_PER_TILE = N_PAD // NSUB

ROWB = 1000
NRB = N // ROWB

@functools.cache
def _mesh():
    return plsc.VectorSubcoreMesh(
        core_axis_name="c", subcore_axis_name="s",
        num_cores=2, num_subcores=NSUB)


def _deg_body(dst_hbm, deg_hbm, dstv, degv, tmpv, outv, shpart):
    c = lax.axis_index("c")
    s = lax.axis_index("s")

    @pl.when(c == 0)
    def _():
        zeros = jnp.zeros((16,), jnp.float32)
        ones = jnp.ones((16,), jnp.float32)

        def zstep(i, carry):
            degv[pl.ds(i * 16, 16)] = zeros
            return carry

        lax.fori_loop(0, N_PAD // 16, zstep, 0)
        pltpu.sync_copy(dst_hbm.at[s], dstv)

        def cstep(k, carry):
            idx = dstv[pl.ds(k * 16, 16)]
            plsc.addupdate_scatter(degv, [idx], ones)
            return carry

        lax.fori_loop(0, EDGES_PER_TILE // 16, cstep, 0)
        pltpu.sync_copy(degv, shpart.at[s])
        plsc.subcore_barrier()

        cslc = pl.ds(s * ROWS_PER_TILE, ROWS_PER_TILE)
        pltpu.sync_copy(shpart.at[0, cslc], outv)
        for r in range(1, NSUB):
            pltpu.sync_copy(shpart.at[r, cslc], tmpv)

            def astep(q, carry):
                sl = pl.ds(q * 16, 16)
                outv[sl] = outv[sl] + tmpv[sl]
                return carry

            lax.fori_loop(0, ROWS_PER_TILE // 16, astep, 0)
        pltpu.sync_copy(outv, deg_hbm.at[cslc])


@functools.cache
def _deg_kernel():
    return pl.kernel(
        _deg_body,
        out_type=jax.ShapeDtypeStruct((N_PAD,), jnp.float32),
        mesh=_mesh(),
        scratch_types=[
            pltpu.VMEM((EDGES_PER_TILE,), jnp.int32),
            pltpu.VMEM((N_PAD,), jnp.float32),
            pltpu.VMEM((ROWS_PER_TILE,), jnp.float32),
            pltpu.VMEM((ROWS_PER_TILE,), jnp.float32),
            pltpu.VMEM_SHARED((NSUB, N_PAD), jnp.float32),
        ],
        compiler_params=pltpu.CompilerParams(needs_layout_passes=False),
    )


def _scatter_half(h_hbm, dst_hbm, acc_hbm, srci_v, dstc_v, rows_a, rows_b,
                  sg, si, ss, shacc, s):
    rslc = pl.ds(s * ROWS_PER_TILE, ROWS_PER_TILE)
    pltpu.sync_copy(h_hbm.at[rslc], shacc.at[rslc])
    plsc.subcore_barrier()

    def gath(j, buf):
        return pltpu.async_copy(
            h_hbm.at[srci_v.at[pl.ds(j * EBATCH, EBATCH)]], buf, sg)

    def gath_wait(j, buf):
        pltpu.make_async_copy(
            h_hbm.at[srci_v.at[pl.ds(j * EBATCH, EBATCH)]], buf, sg).wait()

    def idx(j, p):
        return pltpu.async_copy(dst_hbm.at[s, j], dstc_v.at[p], si)

    def idx_wait(j, p):
        pltpu.make_async_copy(dst_hbm.at[s, j], dstc_v.at[p], si).wait()

    def sct(buf, p):
        return pltpu.async_copy(buf, shacc.at[dstc_v.at[p]], ss, add=True)

    def sct_wait(buf, p):
        pltpu.make_async_copy(buf, shacc.at[dstc_v.at[p]], ss).wait()

    idx(0, 0)
    gath(0, rows_a)
    idx(1, 1)
    gath(1, rows_b)

    def step2(k, carry):
        j = 2 * k
        gath_wait(j, rows_a)
        idx_wait(j, 0)
        sct(rows_a, 0)
        sct_wait(rows_a, 0)

        @pl.when(j + 2 < NEB)
        def _():
            gath(j + 2, rows_a)
            idx(j + 2, 0)

        gath_wait(j + 1, rows_b)
        idx_wait(j + 1, 1)
        sct(rows_b, 1)
        sct_wait(rows_b, 1)

        @pl.when(j + 3 < NEB)
        def _():
            gath(j + 3, rows_b)
            idx(j + 3, 1)

        return carry

    lax.fori_loop(0, NEB // 2, step2, 0)
    plsc.subcore_barrier()
    pltpu.sync_copy(shacc.at[rslc], acc_hbm.at[rslc])


def _gcn_scatter_body(h0, h1, src_hbm, dst_hbm, acc0, acc1,
                      srci_v, dstc_v, rows_a, rows_b, sg, si, ss, shacc):
    c = lax.axis_index("c")
    s = lax.axis_index("s")
    pltpu.sync_copy(src_hbm.at[s], srci_v)

    @pl.when(c == 0)
    def _():
        _scatter_half(h0, dst_hbm, acc0, srci_v, dstc_v, rows_a, rows_b,
                      sg, si, ss, shacc, s)

    @pl.when(c == 1)
    def _():
        _scatter_half(h1, dst_hbm, acc1, srci_v, dstc_v, rows_a, rows_b,
                      sg, si, ss, shacc, s)


@functools.cache
def _gcn_scatter():
    return pl.kernel(
        _gcn_scatter_body,
        out_type=(
            jax.ShapeDtypeStruct((N_PAD, HALF), jnp.float32),
            jax.ShapeDtypeStruct((N_PAD, HALF), jnp.float32),
        ),
        mesh=_mesh(),
        scratch_types=[
            pltpu.VMEM((EPT_PAD,), jnp.int32),
            pltpu.VMEM((2, EBATCH), jnp.int32),
            pltpu.VMEM((EBATCH, HALF), jnp.float32),
            pltpu.VMEM((EBATCH, HALF), jnp.float32),
            pltpu.SemaphoreType.DMA,
            pltpu.SemaphoreType.DMA,
            pltpu.SemaphoreType.DMA,
            pltpu.VMEM_SHARED((N_PAD, HALF), jnp.float32),
        ],
    )


def _mm_body(x_ref, w_ref, deg_ref, h0_ref, h1_ref):
    deg = deg_ref[...] + 1.0
    dinv = 1.0 / jnp.sqrt(deg)
    h = jnp.dot(x_ref[...], w_ref[...], preferred_element_type=jnp.float32)
    h = h * dinv
    h0_ref[...] = h[:, :HALF]
    h1_ref[...] = h[:, HALF:]


def _mm_call(x, w, deg16):
    return pl.pallas_call(
        _mm_body,
        grid=(NRB,),
        in_specs=[
            pl.BlockSpec((ROWB, H), lambda j: (j, 0)),
            pl.BlockSpec((H, H), lambda j: (0, 0)),
            pl.BlockSpec((ROWB, 1), lambda j: (j, 0)),
        ],
        out_specs=(
            pl.BlockSpec((ROWB, HALF), lambda j: (j, 0)),
            pl.BlockSpec((ROWB, HALF), lambda j: (j, 0)),
        ),
        out_shape=(
            jax.ShapeDtypeStruct((N_PAD, HALF), jnp.float32),
            jax.ShapeDtypeStruct((N_PAD, HALF), jnp.float32),
        ),
    )(x, w, deg16)


def _post_body(has_res, a0_ref, a1_ref, deg_ref, b_ref, g_ref, be_ref,
               *rest):
    if has_res:
        res_ref, y_ref, sum_ref, sq_ref = rest
    else:
        y_ref, sum_ref, sq_ref = rest
        res_ref = None
    j = pl.program_id(0)

    @pl.when(j == 0)
    def _():
        sum_ref[...] = jnp.zeros_like(sum_ref)
        sq_ref[...] = jnp.zeros_like(sq_ref)

    def compute_z():
        a = jnp.concatenate([a0_ref[...], a1_ref[...]], axis=1)
        dinv = 1.0 / jnp.sqrt(deg_ref[...] + 1.0)
        z = jnp.maximum(a * dinv + b_ref[...], 0.0)
        if has_res:
            z = jnp.maximum(z + res_ref[...], 0.0)
        return z

    @pl.when(j < NRB)
    def _():
        z = compute_z()
        sum_ref[...] += jnp.sum(z, axis=0, keepdims=True)
        sq_ref[...] += jnp.sum(z * z, axis=0, keepdims=True)

    @pl.when(j >= NRB)
    def _():
        z = compute_z()
        mean = sum_ref[...] * (1.0 / N)
        var = sq_ref[...] * (1.0 / N) - mean * mean
        scale = g_ref[...] / jnp.sqrt(var + EPS)
        y_ref[...] = (z - mean) * scale + be_ref[...]


def _post_call(a0, a1, deg2d, bvec, gvec, bevec, res):
    has_res = res is not None
    in_specs = [
        pl.BlockSpec((ROWB, HALF), lambda j: (j % NRB, 0)),
        pl.BlockSpec((ROWB, HALF), lambda j: (j % NRB, 0)),
        pl.BlockSpec((ROWB, 1), lambda j: (j % NRB, 0)),
        pl.BlockSpec((1, H), lambda j: (0, 0)),
        pl.BlockSpec((1, H), lambda j: (0, 0)),
        pl.BlockSpec((1, H), lambda j: (0, 0)),
    ]
    args = [a0, a1, deg2d, bvec, gvec, bevec]
    if has_res:
        in_specs.append(pl.BlockSpec((ROWB, H), lambda j: (j % NRB, 0)))
        args.append(res)
    return pl.pallas_call(
        functools.partial(_post_body, has_res),
        grid=(2 * NRB,),
        in_specs=in_specs,
        out_specs=pl.BlockSpec((ROWB, H), lambda j: (j % NRB, 0)),
        out_shape=jax.ShapeDtypeStruct((N, H), jnp.float32),
        scratch_shapes=[
            pltpu.VMEM((1, H), jnp.float32),
            pltpu.VMEM((1, H), jnp.float32),
        ],
    )(*args)


def kernel(x, edge_index, W, b, gamma, beta):
    ei = edge_index.astype(jnp.int32)
    pad = EPT_PAD - EDGES_PER_TILE
    src = jnp.pad(ei[0].reshape(NSUB, EDGES_PER_TILE), ((0, 0), (0, pad)),
                  constant_values=0)
    pad_dst = N + (jnp.arange(pad, dtype=jnp.int32)[None, :]
                   + 17 * jnp.arange(NSUB, dtype=jnp.int32)[:, None]) % (N_PAD - N)
    dst = jnp.concatenate(
        [ei[1].reshape(NSUB, EDGES_PER_TILE), pad_dst], axis=1
    ).reshape(NSUB, NEB, EBATCH)

    dst_flat = ei[1].reshape(NSUB, EDGES_PER_TILE)
    deg2d = _deg_kernel()(dst_flat).reshape(N_PAD, 1)

    x0 = x
    cur = x
    for i in range(NUM_LAYERS):
        if i % STEP_LAYER == 0:
            x0 = cur
        x_in = cur
        h0, h1 = _mm_call(cur, W[i], deg2d)
        a0, a1 = _gcn_scatter()(h0, h1, src, dst)
        if i == 0:
            res = None
        elif (i != NUM_LAYERS - 1) and ((i + 1) % STEP_LAYER != 0):
            res = x_in
        else:
            res = x0
        cur = _post_call(a0, a1, deg2d, b[i].reshape(1, H),
                         gamma[i].reshape(1, H), beta[i].reshape(1, H), res)
    return cur


# speedup vs baseline: 1.5683x; 1.0337x over previous
import functools

import jax
import jax.numpy as jnp
from jax import lax
from jax.experimental import pallas as pl
from jax.experimental.pallas import tpu as pltpu
from jax.experimental.pallas import tpu_sc as plsc

N = 10000
N_PAD = 10240
E = 160000
H = 256
HALF = 128
NUM_LAYERS = 6
STEP_LAYER = 3
EPS = 1e-5

NSUB = 16
EDGES_PER_TILE = E // NSUB
EBATCH = 88
EPT_PAD = 10032
NEB = EPT_PAD // EBATCH
ROWS_PER_TILE = N_PAD // NSUB

ROWB = 2000
NRB = N // ROWB

@functools.cache
def _mesh():
    return plsc.VectorSubcoreMesh(
        core_axis_name="c", subcore_axis_name="s",
        num_cores=2, num_subcores=NSUB)


def _deg_body(dst_hbm, deg_hbm, dstv, degv, tmpv, outv, shpart):
    c = lax.axis_index("c")
    s = lax.axis_index("s")

    @pl.when(c == 0)
    def _():
        zeros = jnp.zeros((16,), jnp.float32)
        ones = jnp.ones((16,), jnp.float32)

        def zstep(i, carry):
            degv[pl.ds(i * 16, 16)] = zeros
            return carry

        lax.fori_loop(0, N_PAD // 16, zstep, 0)
        pltpu.sync_copy(dst_hbm.at[s], dstv)

        def cstep(k, carry):
            idx = dstv[pl.ds(k * 16, 16)]
            plsc.addupdate_scatter(degv, [idx], ones)
            return carry

        lax.fori_loop(0, EDGES_PER_TILE // 16, cstep, 0)
        pltpu.sync_copy(degv, shpart.at[s])
        plsc.subcore_barrier()

        cslc = pl.ds(s * ROWS_PER_TILE, ROWS_PER_TILE)
        pltpu.sync_copy(shpart.at[0, cslc], outv)
        for r in range(1, NSUB):
            pltpu.sync_copy(shpart.at[r, cslc], tmpv)

            def astep(q, carry):
                sl = pl.ds(q * 16, 16)
                outv[sl] = outv[sl] + tmpv[sl]
                return carry

            lax.fori_loop(0, ROWS_PER_TILE // 16, astep, 0)
        pltpu.sync_copy(outv, deg_hbm.at[cslc])


@functools.cache
def _deg_kernel():
    return pl.kernel(
        _deg_body,
        out_type=jax.ShapeDtypeStruct((N_PAD,), jnp.float32),
        mesh=_mesh(),
        scratch_types=[
            pltpu.VMEM((EDGES_PER_TILE,), jnp.int32),
            pltpu.VMEM((N_PAD,), jnp.float32),
            pltpu.VMEM((ROWS_PER_TILE,), jnp.float32),
            pltpu.VMEM((ROWS_PER_TILE,), jnp.float32),
            pltpu.VMEM_SHARED((NSUB, N_PAD), jnp.float32),
        ],
        compiler_params=pltpu.CompilerParams(needs_layout_passes=False),
    )


def _scatter_half(h_hbm, dst_hbm, acc_hbm, srci_v, dstc_v, rows_a, rows_b,
                  sg, si, ss, shacc, s):
    rslc = pl.ds(s * ROWS_PER_TILE, ROWS_PER_TILE)
    pltpu.sync_copy(h_hbm.at[rslc], shacc.at[rslc])
    plsc.subcore_barrier()

    def gath(j, buf):
        return pltpu.async_copy(
            h_hbm.at[srci_v.at[pl.ds(j * EBATCH, EBATCH)]], buf, sg)

    def gath_wait(j, buf):
        pltpu.make_async_copy(
            h_hbm.at[srci_v.at[pl.ds(j * EBATCH, EBATCH)]], buf, sg).wait()

    def idx(j, p):
        return pltpu.async_copy(dst_hbm.at[s, j], dstc_v.at[p], si)

    def idx_wait(j, p):
        pltpu.make_async_copy(dst_hbm.at[s, j], dstc_v.at[p], si).wait()

    def sct(buf, p):
        return pltpu.async_copy(buf, shacc.at[dstc_v.at[p]], ss, add=True)

    def sct_wait(buf, p):
        pltpu.make_async_copy(buf, shacc.at[dstc_v.at[p]], ss).wait()

    idx(0, 0)
    gath(0, rows_a)
    idx(1, 1)
    gath(1, rows_b)

    def step2(k, carry):
        j = 2 * k
        gath_wait(j, rows_a)
        idx_wait(j, 0)
        sct(rows_a, 0)
        sct_wait(rows_a, 0)

        @pl.when(j + 2 < NEB)
        def _():
            gath(j + 2, rows_a)
            idx(j + 2, 0)

        gath_wait(j + 1, rows_b)
        idx_wait(j + 1, 1)
        sct(rows_b, 1)
        sct_wait(rows_b, 1)

        @pl.when(j + 3 < NEB)
        def _():
            gath(j + 3, rows_b)
            idx(j + 3, 1)

        return carry

    lax.fori_loop(0, NEB // 2, step2, 0)
    plsc.subcore_barrier()
    pltpu.sync_copy(shacc.at[rslc], acc_hbm.at[rslc])


def _gcn_scatter_body(h0, h1, src_hbm, dst_hbm, acc0, acc1,
                      srci_v, dstc_v, rows_a, rows_b, sg, si, ss, shacc):
    c = lax.axis_index("c")
    s = lax.axis_index("s")
    pltpu.sync_copy(src_hbm.at[s], srci_v)

    @pl.when(c == 0)
    def _():
        _scatter_half(h0, dst_hbm, acc0, srci_v, dstc_v, rows_a, rows_b,
                      sg, si, ss, shacc, s)

    @pl.when(c == 1)
    def _():
        _scatter_half(h1, dst_hbm, acc1, srci_v, dstc_v, rows_a, rows_b,
                      sg, si, ss, shacc, s)


@functools.cache
def _gcn_scatter():
    return pl.kernel(
        _gcn_scatter_body,
        out_type=(
            jax.ShapeDtypeStruct((N_PAD, HALF), jnp.float32),
            jax.ShapeDtypeStruct((N_PAD, HALF), jnp.float32),
        ),
        mesh=_mesh(),
        scratch_types=[
            pltpu.VMEM((EPT_PAD,), jnp.int32),
            pltpu.VMEM((2, EBATCH), jnp.int32),
            pltpu.VMEM((EBATCH, HALF), jnp.float32),
            pltpu.VMEM((EBATCH, HALF), jnp.float32),
            pltpu.SemaphoreType.DMA,
            pltpu.SemaphoreType.DMA,
            pltpu.SemaphoreType.DMA,
            pltpu.VMEM_SHARED((N_PAD, HALF), jnp.float32),
        ],
    )


def _mm_body(x_ref, w_ref, deg_ref, h0_ref, h1_ref):
    deg = deg_ref[...] + 1.0
    dinv = 1.0 / jnp.sqrt(deg)
    h = jnp.dot(x_ref[...], w_ref[...], preferred_element_type=jnp.float32)
    h = h * dinv
    h0_ref[...] = h[:, :HALF]
    h1_ref[...] = h[:, HALF:]


def _mm_call(x, w, deg16):
    return pl.pallas_call(
        _mm_body,
        grid=(NRB,),
        in_specs=[
            pl.BlockSpec((ROWB, H), lambda j: (j, 0)),
            pl.BlockSpec((H, H), lambda j: (0, 0)),
            pl.BlockSpec((ROWB, 1), lambda j: (j, 0)),
        ],
        out_specs=(
            pl.BlockSpec((ROWB, HALF), lambda j: (j, 0)),
            pl.BlockSpec((ROWB, HALF), lambda j: (j, 0)),
        ),
        out_shape=(
            jax.ShapeDtypeStruct((N_PAD, HALF), jnp.float32),
            jax.ShapeDtypeStruct((N_PAD, HALF), jnp.float32),
        ),
    )(x, w, deg16)


def _post_body(has_res, a0_ref, a1_ref, deg_ref, b_ref, g_ref, be_ref,
               *rest):
    if has_res:
        res_ref, y_ref, sum_ref, sq_ref = rest
    else:
        y_ref, sum_ref, sq_ref = rest
        res_ref = None
    j = pl.program_id(0)

    @pl.when(j == 0)
    def _():
        sum_ref[...] = jnp.zeros_like(sum_ref)
        sq_ref[...] = jnp.zeros_like(sq_ref)

    def compute_z():
        a = jnp.concatenate([a0_ref[...], a1_ref[...]], axis=1)
        dinv = 1.0 / jnp.sqrt(deg_ref[...] + 1.0)
        z = jnp.maximum(a * dinv + b_ref[...], 0.0)
        if has_res:
            z = jnp.maximum(z + res_ref[...], 0.0)
        return z

    @pl.when(j < NRB)
    def _():
        z = compute_z()
        sum_ref[...] += jnp.sum(z, axis=0, keepdims=True)
        sq_ref[...] += jnp.sum(z * z, axis=0, keepdims=True)

    @pl.when(j >= NRB)
    def _():
        z = compute_z()
        mean = sum_ref[...] * (1.0 / N)
        var = sq_ref[...] * (1.0 / N) - mean * mean
        scale = g_ref[...] / jnp.sqrt(var + EPS)
        y_ref[...] = (z - mean) * scale + be_ref[...]


def _post_call(a0, a1, deg2d, bvec, gvec, bevec, res):
    has_res = res is not None
    in_specs = [
        pl.BlockSpec((ROWB, HALF), lambda j: (j % NRB, 0)),
        pl.BlockSpec((ROWB, HALF), lambda j: (j % NRB, 0)),
        pl.BlockSpec((ROWB, 1), lambda j: (j % NRB, 0)),
        pl.BlockSpec((1, H), lambda j: (0, 0)),
        pl.BlockSpec((1, H), lambda j: (0, 0)),
        pl.BlockSpec((1, H), lambda j: (0, 0)),
    ]
    args = [a0, a1, deg2d, bvec, gvec, bevec]
    if has_res:
        in_specs.append(pl.BlockSpec((ROWB, H), lambda j: (j % NRB, 0)))
        args.append(res)
    return pl.pallas_call(
        functools.partial(_post_body, has_res),
        grid=(2 * NRB,),
        in_specs=in_specs,
        out_specs=pl.BlockSpec((ROWB, H), lambda j: (j % NRB, 0)),
        out_shape=jax.ShapeDtypeStruct((N, H), jnp.float32),
        scratch_shapes=[
            pltpu.VMEM((1, H), jnp.float32),
            pltpu.VMEM((1, H), jnp.float32),
        ],
    )(*args)


def kernel(x, edge_index, W, b, gamma, beta):
    ei = edge_index.astype(jnp.int32)
    pad = EPT_PAD - EDGES_PER_TILE
    src = jnp.pad(ei[0].reshape(NSUB, EDGES_PER_TILE), ((0, 0), (0, pad)),
                  constant_values=0)
    pad_dst = N + (jnp.arange(pad, dtype=jnp.int32)[None, :]
                   + 17 * jnp.arange(NSUB, dtype=jnp.int32)[:, None]) % (N_PAD - N)
    dst = jnp.concatenate(
        [ei[1].reshape(NSUB, EDGES_PER_TILE), pad_dst], axis=1
    ).reshape(NSUB, NEB, EBATCH)

    dst_flat = ei[1].reshape(NSUB, EDGES_PER_TILE)
    deg2d = _deg_kernel()(dst_flat).reshape(N_PAD, 1)

    x0 = x
    cur = x
    for i in range(NUM_LAYERS):
        if i % STEP_LAYER == 0:
            x0 = cur
        x_in = cur
        h0, h1 = _mm_call(cur, W[i], deg2d)
        a0, a1 = _gcn_scatter()(h0, h1, src, dst)
        if i == 0:
            res = None
        elif (i != NUM_LAYERS - 1) and ((i + 1) % STEP_LAYER != 0):
            res = x_in
        else:
            res = x0
        cur = _post_call(a0, a1, deg2d, b[i].reshape(1, H),
                         gamma[i].reshape(1, H), beta[i].reshape(1, H), res)
    return cur


# speedup vs baseline: 1.5953x; 1.0172x over previous
import functools

import jax
import jax.numpy as jnp
from jax import lax
from jax.experimental import pallas as pl
from jax.experimental.pallas import tpu as pltpu
from jax.experimental.pallas import tpu_sc as plsc

N = 10000
N_PAD = 10240
E = 160000
H = 256
HALF = 128
NUM_LAYERS = 6
STEP_LAYER = 3
EPS = 1e-5

NSUB = 16
EDGES_PER_TILE = E // NSUB
EBATCH = 88
EPT_PAD = 10032
NEB = EPT_PAD // EBATCH
ROWS_PER_TILE = N_PAD // NSUB

ROWB = 5000
NRB = N // ROWB

@functools.cache
def _mesh():
    return plsc.VectorSubcoreMesh(
        core_axis_name="c", subcore_axis_name="s",
        num_cores=2, num_subcores=NSUB)


def _deg_body(dst_hbm, deg_hbm, dstv, degv, tmpv, outv, shpart):
    c = lax.axis_index("c")
    s = lax.axis_index("s")

    @pl.when(c == 0)
    def _():
        zeros = jnp.zeros((16,), jnp.float32)
        ones = jnp.ones((16,), jnp.float32)

        def zstep(i, carry):
            degv[pl.ds(i * 16, 16)] = zeros
            return carry

        lax.fori_loop(0, N_PAD // 16, zstep, 0)
        pltpu.sync_copy(dst_hbm.at[s], dstv)

        def cstep(k, carry):
            idx = dstv[pl.ds(k * 16, 16)]
            plsc.addupdate_scatter(degv, [idx], ones)
            return carry

        lax.fori_loop(0, EDGES_PER_TILE // 16, cstep, 0)
        pltpu.sync_copy(degv, shpart.at[s])
        plsc.subcore_barrier()

        cslc = pl.ds(s * ROWS_PER_TILE, ROWS_PER_TILE)
        pltpu.sync_copy(shpart.at[0, cslc], outv)
        for r in range(1, NSUB):
            pltpu.sync_copy(shpart.at[r, cslc], tmpv)

            def astep(q, carry):
                sl = pl.ds(q * 16, 16)
                outv[sl] = outv[sl] + tmpv[sl]
                return carry

            lax.fori_loop(0, ROWS_PER_TILE // 16, astep, 0)
        pltpu.sync_copy(outv, deg_hbm.at[cslc])


@functools.cache
def _deg_kernel():
    return pl.kernel(
        _deg_body,
        out_type=jax.ShapeDtypeStruct((N_PAD,), jnp.float32),
        mesh=_mesh(),
        scratch_types=[
            pltpu.VMEM((EDGES_PER_TILE,), jnp.int32),
            pltpu.VMEM((N_PAD,), jnp.float32),
            pltpu.VMEM((ROWS_PER_TILE,), jnp.float32),
            pltpu.VMEM((ROWS_PER_TILE,), jnp.float32),
            pltpu.VMEM_SHARED((NSUB, N_PAD), jnp.float32),
        ],
        compiler_params=pltpu.CompilerParams(needs_layout_passes=False),
    )


def _scatter_half(h_hbm, dst_hbm, acc_hbm, srci_v, dstc_v, rows_a, rows_b,
                  sg, si, ss, shacc, s):
    rslc = pl.ds(s * ROWS_PER_TILE, ROWS_PER_TILE)
    pltpu.sync_copy(h_hbm.at[rslc], shacc.at[rslc])
    plsc.subcore_barrier()

    def gath(j, buf):
        return pltpu.async_copy(
            h_hbm.at[srci_v.at[pl.ds(j * EBATCH, EBATCH)]], buf, sg)

    def gath_wait(j, buf):
        pltpu.make_async_copy(
            h_hbm.at[srci_v.at[pl.ds(j * EBATCH, EBATCH)]], buf, sg).wait()

    def idx(j, p):
        return pltpu.async_copy(dst_hbm.at[s, j], dstc_v.at[p], si)

    def idx_wait(j, p):
        pltpu.make_async_copy(dst_hbm.at[s, j], dstc_v.at[p], si).wait()

    def sct(buf, p):
        return pltpu.async_copy(buf, shacc.at[dstc_v.at[p]], ss, add=True)

    def sct_wait(buf, p):
        pltpu.make_async_copy(buf, shacc.at[dstc_v.at[p]], ss).wait()

    idx(0, 0)
    gath(0, rows_a)
    idx(1, 1)
    gath(1, rows_b)

    def step2(k, carry):
        j = 2 * k
        gath_wait(j, rows_a)
        idx_wait(j, 0)
        sct(rows_a, 0)
        sct_wait(rows_a, 0)

        @pl.when(j + 2 < NEB)
        def _():
            gath(j + 2, rows_a)
            idx(j + 2, 0)

        gath_wait(j + 1, rows_b)
        idx_wait(j + 1, 1)
        sct(rows_b, 1)
        sct_wait(rows_b, 1)

        @pl.when(j + 3 < NEB)
        def _():
            gath(j + 3, rows_b)
            idx(j + 3, 1)

        return carry

    lax.fori_loop(0, NEB // 2, step2, 0)
    plsc.subcore_barrier()
    pltpu.sync_copy(shacc.at[rslc], acc_hbm.at[rslc])


def _gcn_scatter_body(h0, h1, src_hbm, dst_hbm, acc0, acc1,
                      srci_v, dstc_v, rows_a, rows_b, sg, si, ss, shacc):
    c = lax.axis_index("c")
    s = lax.axis_index("s")
    pltpu.sync_copy(src_hbm.at[s], srci_v)

    @pl.when(c == 0)
    def _():
        _scatter_half(h0, dst_hbm, acc0, srci_v, dstc_v, rows_a, rows_b,
                      sg, si, ss, shacc, s)

    @pl.when(c == 1)
    def _():
        _scatter_half(h1, dst_hbm, acc1, srci_v, dstc_v, rows_a, rows_b,
                      sg, si, ss, shacc, s)


@functools.cache
def _gcn_scatter():
    return pl.kernel(
        _gcn_scatter_body,
        out_type=(
            jax.ShapeDtypeStruct((N_PAD, HALF), jnp.float32),
            jax.ShapeDtypeStruct((N_PAD, HALF), jnp.float32),
        ),
        mesh=_mesh(),
        scratch_types=[
            pltpu.VMEM((EPT_PAD,), jnp.int32),
            pltpu.VMEM((2, EBATCH), jnp.int32),
            pltpu.VMEM((EBATCH, HALF), jnp.float32),
            pltpu.VMEM((EBATCH, HALF), jnp.float32),
            pltpu.SemaphoreType.DMA,
            pltpu.SemaphoreType.DMA,
            pltpu.SemaphoreType.DMA,
            pltpu.VMEM_SHARED((N_PAD, HALF), jnp.float32),
        ],
    )


def _mm_body(x_ref, w_ref, deg_ref, h0_ref, h1_ref):
    deg = deg_ref[...] + 1.0
    dinv = 1.0 / jnp.sqrt(deg)
    h = jnp.dot(x_ref[...], w_ref[...], preferred_element_type=jnp.float32)
    h = h * dinv
    h0_ref[...] = h[:, :HALF]
    h1_ref[...] = h[:, HALF:]


def _mm_call(x, w, deg16):
    return pl.pallas_call(
        _mm_body,
        grid=(NRB,),
        in_specs=[
            pl.BlockSpec((ROWB, H), lambda j: (j, 0)),
            pl.BlockSpec((H, H), lambda j: (0, 0)),
            pl.BlockSpec((ROWB, 1), lambda j: (j, 0)),
        ],
        out_specs=(
            pl.BlockSpec((ROWB, HALF), lambda j: (j, 0)),
            pl.BlockSpec((ROWB, HALF), lambda j: (j, 0)),
        ),
        out_shape=(
            jax.ShapeDtypeStruct((N_PAD, HALF), jnp.float32),
            jax.ShapeDtypeStruct((N_PAD, HALF), jnp.float32),
        ),
    )(x, w, deg16)


def _post_body(has_res, a0_ref, a1_ref, deg_ref, b_ref, g_ref, be_ref,
               *rest):
    if has_res:
        res_ref, y_ref, sum_ref, sq_ref = rest
    else:
        y_ref, sum_ref, sq_ref = rest
        res_ref = None
    j = pl.program_id(0)

    @pl.when(j == 0)
    def _():
        sum_ref[...] = jnp.zeros_like(sum_ref)
        sq_ref[...] = jnp.zeros_like(sq_ref)

    def compute_z():
        a = jnp.concatenate([a0_ref[...], a1_ref[...]], axis=1)
        dinv = 1.0 / jnp.sqrt(deg_ref[...] + 1.0)
        z = jnp.maximum(a * dinv + b_ref[...], 0.0)
        if has_res:
            z = jnp.maximum(z + res_ref[...], 0.0)
        return z

    @pl.when(j < NRB)
    def _():
        z = compute_z()
        sum_ref[...] += jnp.sum(z, axis=0, keepdims=True)
        sq_ref[...] += jnp.sum(z * z, axis=0, keepdims=True)

    @pl.when(j >= NRB)
    def _():
        z = compute_z()
        mean = sum_ref[...] * (1.0 / N)
        var = sq_ref[...] * (1.0 / N) - mean * mean
        scale = g_ref[...] / jnp.sqrt(var + EPS)
        y_ref[...] = (z - mean) * scale + be_ref[...]


def _post_call(a0, a1, deg2d, bvec, gvec, bevec, res):
    has_res = res is not None
    in_specs = [
        pl.BlockSpec((ROWB, HALF), lambda j: (j % NRB, 0)),
        pl.BlockSpec((ROWB, HALF), lambda j: (j % NRB, 0)),
        pl.BlockSpec((ROWB, 1), lambda j: (j % NRB, 0)),
        pl.BlockSpec((1, H), lambda j: (0, 0)),
        pl.BlockSpec((1, H), lambda j: (0, 0)),
        pl.BlockSpec((1, H), lambda j: (0, 0)),
    ]
    args = [a0, a1, deg2d, bvec, gvec, bevec]
    if has_res:
        in_specs.append(pl.BlockSpec((ROWB, H), lambda j: (j % NRB, 0)))
        args.append(res)
    return pl.pallas_call(
        functools.partial(_post_body, has_res),
        grid=(2 * NRB,),
        in_specs=in_specs,
        out_specs=pl.BlockSpec((ROWB, H), lambda j: (j % NRB, 0)),
        out_shape=jax.ShapeDtypeStruct((N, H), jnp.float32),
        scratch_shapes=[
            pltpu.VMEM((1, H), jnp.float32),
            pltpu.VMEM((1, H), jnp.float32),
        ],
    )(*args)


def kernel(x, edge_index, W, b, gamma, beta):
    ei = edge_index.astype(jnp.int32)
    pad = EPT_PAD - EDGES_PER_TILE
    src = jnp.pad(ei[0].reshape(NSUB, EDGES_PER_TILE), ((0, 0), (0, pad)),
                  constant_values=0)
    pad_dst = N + (jnp.arange(pad, dtype=jnp.int32)[None, :]
                   + 17 * jnp.arange(NSUB, dtype=jnp.int32)[:, None]) % (N_PAD - N)
    dst = jnp.concatenate(
        [ei[1].reshape(NSUB, EDGES_PER_TILE), pad_dst], axis=1
    ).reshape(NSUB, NEB, EBATCH)

    dst_flat = ei[1].reshape(NSUB, EDGES_PER_TILE)
    deg2d = _deg_kernel()(dst_flat).reshape(N_PAD, 1)

    x0 = x
    cur = x
    for i in range(NUM_LAYERS):
        if i % STEP_LAYER == 0:
            x0 = cur
        x_in = cur
        h0, h1 = _mm_call(cur, W[i], deg2d)
        a0, a1 = _gcn_scatter()(h0, h1, src, dst)
        if i == 0:
            res = None
        elif (i != NUM_LAYERS - 1) and ((i + 1) % STEP_LAYER != 0):
            res = x_in
        else:
            res = x0
        cur = _post_call(a0, a1, deg2d, b[i].reshape(1, H),
                         gamma[i].reshape(1, H), beta[i].reshape(1, H), res)
    return cur
